# baseline (speedup 1.0000x reference)
import jax
import jax.numpy as jnp
from jax import lax
from jax.experimental import pallas as pl
from jax.experimental.pallas import tpu as pltpu
from jax.experimental.pallas import tpu_sc as plsc

B = 16384
D = 64
NC = 2
NS = 16
L = 16
NW = NC * NS
BPW = B // NW
CH = 128
NJ = BPW // CH

_params = pltpu.CompilerParams(
    needs_layout_passes=False, use_tc_tiling_on_sc=False)
_mesh = plsc.VectorSubcoreMesh(core_axis_name="c", subcore_axis_name="s")


def _wid():
    return lax.axis_index("s") * NC + lax.axis_index("c")


def _row_gather_body(idx_hbm, table_hbm, out_hbm, idx_v, rows_v, sem):
    wid = _wid()
    pltpu.sync_copy(idx_hbm.at[wid], idx_v)
    for j in range(NJ):
        pltpu.async_copy(table_hbm.at[idx_v.at[j]],
                         rows_v.at[pl.ds(j * CH, CH)], sem)
    pltpu.make_async_copy(table_hbm.at[pl.ds(0, BPW)], rows_v, sem).wait()
    pltpu.sync_copy(rows_v, out_hbm.at[pl.ds(wid * BPW, BPW)])


def _val_gather_body(idx_hbm, vals_hbm, out_hbm, idx_v, vals_v, sem):
    wid = _wid()
    pltpu.sync_copy(idx_hbm.at[wid], idx_v)
    for j in range(NJ):
        pltpu.async_copy(vals_hbm.at[idx_v.at[j]],
                         vals_v.at[pl.ds(j * CH, CH)], sem)
    pltpu.make_async_copy(vals_hbm.at[pl.ds(0, BPW)], vals_v, sem).wait()
    pltpu.sync_copy(vals_v, out_hbm.at[pl.ds(wid * BPW, BPW)])


def _dot_body(urows_hbm, mrows_hbm, ubv_hbm, mbv_hbm, out_hbm,
              u_v, m_v, ub_v, mb_v, out_v, sem):
    wid = _wid()
    base = wid * BPW
    pltpu.sync_copy(urows_hbm.at[pl.ds(base, BPW)], u_v)
    pltpu.sync_copy(mrows_hbm.at[pl.ds(base, BPW)], m_v)
    pltpu.sync_copy(ubv_hbm.at[pl.ds(base, BPW)], ub_v)
    pltpu.sync_copy(mbv_hbm.at[pl.ds(base, BPW)], mb_v)

    lane = lax.iota(jnp.int32, L)

    def group(g, carry):
        pv = g * L + lane
        acc = ub_v[pl.ds(g * L, L)] + mb_v[pl.ds(g * L, L)]
        for k in range(D):
            kv = jnp.full((L,), k, jnp.int32)
            u = plsc.load_gather(u_v, [pv, kv])
            m = plsc.load_gather(m_v, [pv, kv])
            acc = acc + u * m
        out_v[pl.ds(g * L, L)] = acc
        return carry

    lax.fori_loop(0, BPW // L, group, 0)
    pltpu.sync_copy(out_v, out_hbm.at[pl.ds(base, BPW)])


def kernel(user_ids, movie_ids, users, movies, user_bias, movie_bias):
    uid = user_ids.astype(jnp.int32).reshape(NW, NJ, CH)
    mid = movie_ids.astype(jnp.int32).reshape(NW, NJ, CH)
    ubf = user_bias.reshape(-1)
    mbf = movie_bias.reshape(-1)

    def row_gather(table):
        return pl.kernel(
            _row_gather_body,
            out_type=jax.ShapeDtypeStruct((B, D), jnp.float32),
            mesh=_mesh,
            compiler_params=_params,
            scratch_types=[
                pltpu.VMEM((NJ, CH), jnp.int32),
                pltpu.VMEM((BPW, D), jnp.float32),
                pltpu.SemaphoreType.DMA,
            ],
        )

    def val_gather():
        return pl.kernel(
            _val_gather_body,
            out_type=jax.ShapeDtypeStruct((B,), jnp.float32),
            mesh=_mesh,
            compiler_params=_params,
            scratch_types=[
                pltpu.VMEM((NJ, CH), jnp.int32),
                pltpu.VMEM((BPW,), jnp.float32),
                pltpu.SemaphoreType.DMA,
            ],
        )

    urows = row_gather(users)(uid, users)
    mrows = row_gather(movies)(mid, movies)
    ubv = val_gather()(uid, ubf)
    mbv = val_gather()(mid, mbf)

    dot = pl.kernel(
        _dot_body,
        out_type=jax.ShapeDtypeStruct((B,), jnp.float32),
        mesh=_mesh,
        compiler_params=_params,
        scratch_types=[
            pltpu.VMEM((BPW, D), jnp.float32),
            pltpu.VMEM((BPW, D), jnp.float32),
            pltpu.VMEM((BPW,), jnp.float32),
            pltpu.VMEM((BPW,), jnp.float32),
            pltpu.VMEM((BPW,), jnp.float32),
            pltpu.SemaphoreType.DMA,
        ],
    )
    return dot(urows, mrows, ubv, mbv)


# speedup vs baseline: 1.0146x; 1.0146x over previous
import jax
import jax.numpy as jnp
from jax import lax
from jax.experimental import pallas as pl
from jax.experimental.pallas import tpu as pltpu
from jax.experimental.pallas import tpu_sc as plsc

B = 16384
D = 64
NC = 2
NS = 16
L = 16
NW = NC * NS
BPW = B // NW
CH = 128
NJ = BPW // CH


def _fac_body(uid_hbm, mid_hbm, users_hbm, movies_hbm, ub_hbm, mb_hbm,
              out_hbm,
              uidx_v, midx_v, urows_v, mrows_v, ub_v, mb_v, out_v, sem):
    c = lax.axis_index("c")
    s = lax.axis_index("s")
    wid = s * NC + c

    pltpu.sync_copy(uid_hbm.at[wid], uidx_v)
    pltpu.sync_copy(mid_hbm.at[wid], midx_v)

    for j in range(NJ):
        rsl = pl.ds(j * CH, CH)
        pltpu.async_copy(users_hbm.at[uidx_v.at[j]], urows_v.at[rsl], sem)
        pltpu.async_copy(movies_hbm.at[midx_v.at[j]], mrows_v.at[rsl], sem)
        pltpu.async_copy(ub_hbm.at[uidx_v.at[j]], ub_v.at[rsl], sem)
        pltpu.async_copy(mb_hbm.at[midx_v.at[j]], mb_v.at[rsl], sem)
    pltpu.make_async_copy(users_hbm.at[pl.ds(0, BPW)], urows_v, sem).wait()
    pltpu.make_async_copy(movies_hbm.at[pl.ds(0, BPW)], mrows_v, sem).wait()
    pltpu.make_async_copy(ub_hbm.at[pl.ds(0, BPW)], ub_v, sem).wait()
    pltpu.make_async_copy(mb_hbm.at[pl.ds(0, BPW)], mb_v, sem).wait()

    lane = lax.iota(jnp.int32, L)

    def group(g, carry):
        pv = g * L + lane
        acc = ub_v[pl.ds(g * L, L)] + mb_v[pl.ds(g * L, L)]
        for k in range(D):
            kv = jnp.full((L,), k, jnp.int32)
            u = plsc.load_gather(urows_v, [pv, kv])
            m = plsc.load_gather(mrows_v, [pv, kv])
            acc = acc + u * m
        out_v[pl.ds(g * L, L)] = acc
        return carry

    lax.fori_loop(0, BPW // L, group, 0)

    pltpu.sync_copy(out_v, out_hbm.at[pl.ds(wid * BPW, BPW)])


def kernel(user_ids, movie_ids, users, movies, user_bias, movie_bias):
    uid = user_ids.astype(jnp.int32).reshape(NW, NJ, CH)
    mid = movie_ids.astype(jnp.int32).reshape(NW, NJ, CH)
    ubf = user_bias.reshape(-1)
    mbf = movie_bias.reshape(-1)

    mesh = plsc.VectorSubcoreMesh(core_axis_name="c", subcore_axis_name="s")
    fn = pl.kernel(
        _fac_body,
        out_type=jax.ShapeDtypeStruct((B,), jnp.float32),
        mesh=mesh,
        compiler_params=pltpu.CompilerParams(
            needs_layout_passes=False, use_tc_tiling_on_sc=False),
        scratch_types=[
            pltpu.VMEM((NJ, CH), jnp.int32),
            pltpu.VMEM((NJ, CH), jnp.int32),
            pltpu.VMEM((BPW, D), jnp.float32),
            pltpu.VMEM((BPW, D), jnp.float32),
            pltpu.VMEM((BPW,), jnp.float32),
            pltpu.VMEM((BPW,), jnp.float32),
            pltpu.VMEM((BPW,), jnp.float32),
            pltpu.SemaphoreType.DMA,
        ],
    )
    return fn(uid, mid, users, movies, ubf, mbf)
